# baseline (speedup 1.0000x reference)
import functools

import jax
import jax.numpy as jnp
from jax import lax
from jax.experimental import pallas as pl
from jax.experimental.pallas import tpu as pltpu
from jax.experimental.pallas import tpu_sc as plsc

N = 10000
E = 320000
D_IN = 128
D_HID = 64
D_OUT = 16
PI = 0.5
SIGMA_1 = 1.0
SIGMA_2 = 0.0025

NC = 2
NS = 16
NW = NC * NS
LANES = 16
K = 128
C = 2 * (-(-E // (NW * K * 2)))
EW_PAD = C * K
E_PAD = NW * EW_PAD
N_PAD = 10240
ROWS_PER_TILE = N_PAD // NS
CB1 = 2
CB2 = 8


def _deg_body(dst_hbm, ew_hbm, degp_hbm, dst_v, ew_v, deg_v):
  cid = lax.axis_index("c")
  sid = lax.axis_index("s")
  wid = cid * NS + sid
  pltpu.sync_copy(dst_hbm.at[wid], dst_v)
  pltpu.sync_copy(ew_hbm.at[wid], ew_v)

  def zero(i, _):
    deg_v[pl.ds(i * LANES, LANES)] = jnp.zeros((LANES,), jnp.float32)
    return _
  lax.fori_loop(0, N_PAD // LANES, zero, None)

  def acc(i, _):
    sl = pl.ds(i * LANES, LANES)
    plsc.addupdate_scatter(deg_v, [dst_v[sl]], ew_v[sl])
    return _
  lax.fori_loop(0, EW_PAD // LANES, acc, None)
  pltpu.sync_copy(deg_v, degp_hbm.at[wid])


def _sc_deg(dst_flat, ew_flat):
  mesh = plsc.VectorSubcoreMesh(core_axis_name="c", subcore_axis_name="s")
  fn = pl.kernel(
      _deg_body,
      out_type=jax.ShapeDtypeStruct((NW, N_PAD), jnp.float32),
      mesh=mesh,
      compiler_params=pltpu.CompilerParams(needs_layout_passes=False, use_tc_tiling_on_sc=False),
      scratch_types=[
          pltpu.VMEM((EW_PAD,), jnp.int32),
          pltpu.VMEM((EW_PAD,), jnp.float32),
          pltpu.VMEM((N_PAD,), jnp.float32),
      ],
  )
  return fn(dst_flat.reshape(NW, EW_PAD), ew_flat.reshape(NW, EW_PAD))


def _log_gauss_const(sigma):
  return -0.5 * jnp.log(2.0 * jnp.pi) - jnp.log(sigma)


def _tc1_body(degp, x, w1, rho_w1, b1, rho_b1, w2, rho_w2, b2, rho_b2,
              dinv_o, xw_o, logp_o, logq_o):
  deg = jnp.sum(degp[...], axis=0)
  dinv = jnp.where(deg > 0, lax.rsqrt(deg), 0.0)
  dinv_o[...] = dinv[None, :]
  xw_o[...] = jnp.dot(x[...], w1[...], preferred_element_type=jnp.float32)

  def log_prior(w):
    lg1 = _log_gauss_const(SIGMA_1) - w * w / (2.0 * SIGMA_1 ** 2)
    lg2 = _log_gauss_const(SIGMA_2) - w * w / (2.0 * SIGMA_2 ** 2)
    p = PI * jnp.exp(lg1) + (1.0 - PI) * jnp.exp(lg2)
    return jnp.sum(jnp.log(p + 1e-30))

  def log_q(rho):
    s = jnp.log1p(jnp.exp(rho))
    return jnp.sum(-0.5 * jnp.log(2.0 * jnp.pi) - jnp.log(s))

  logp = (log_prior(w1[...]) + log_prior(b1[...])
          + log_prior(w2[...]) + log_prior(b2[...]))
  logq = (log_q(rho_w1[...]) + log_q(rho_b1[...])
          + log_q(rho_w2[...]) + log_q(rho_b2[...]))
  logp_o[...] = logp[None, None]
  logq_o[...] = logq[None, None]


def _ring_loop(table_hbm, src_v, dst_v, norm_v, rows, gsem, ssem, acc, d, cb):
  nb = C // cb
  vecs = cb * K // LANES

  pltpu.async_copy(table_hbm.at[src_v.at[pl.ds(0, cb * K)]], rows[0], gsem[0])

  def it_body(g, _):
    for b in range(2):
      gi = g * 2 + b
      base = gi * cb
      off = base * K
      pltpu.make_async_copy(
          table_hbm.at[src_v.at[pl.ds(off, cb * K)]], rows[b], gsem[b]).wait()

      @pl.when(gi > 0)
      def _wait_prev_scatter():
        pltpu.make_async_copy(
            rows[1 - b], acc.at[dst_v.at[pl.ds(off - cb * K, cb * K)]],
            ssem[1 - b]).wait()

      @pl.when(gi + 1 < nb)
      def _prefetch_next():
        pltpu.async_copy(
            table_hbm.at[src_v.at[pl.ds(off + cb * K, cb * K)]], rows[1 - b],
            gsem[1 - b])

      def scale(j, _):
        nv = norm_v[pl.ds((gi * vecs + j) * LANES, LANES)]
        for l in range(LANES):
          k = j * LANES + l
          s = nv[l]
          for f in range(d // LANES):
            sl = pl.ds(f * LANES, LANES)
            rows[b][k, sl] = rows[b][k, sl] * s
        return _
      lax.fori_loop(0, vecs, scale, None)
      pltpu.async_copy(rows[b], acc.at[dst_v.at[pl.ds(off, cb * K)]], ssem[b],
                       add=True)
    return _
  lax.fori_loop(0, nb // 2, it_body, None)
  lb = (nb - 1) % 2
  pltpu.make_async_copy(
      rows[lb], acc.at[dst_v.at[pl.ds((C - cb) * K, cb * K)]], ssem[lb]).wait()


def _prop1_body(xw_hbm, src_hbm, dst_hbm, ew_hbm, dinv_hbm,
                hpart_hbm, norm_hbm,
                src_v, dst_v, ew_v, norm_v, dinv_v,
                rows0_v, rows1_v,
                gsem0, gsem1, ssem0, ssem1, acc):
  cid = lax.axis_index("c")
  sid = lax.axis_index("s")
  wid = cid * NS + sid
  rows = (rows0_v, rows1_v)
  gsem = (gsem0, gsem1)
  ssem = (ssem0, ssem1)
  pltpu.sync_copy(src_hbm.at[wid], src_v)
  pltpu.sync_copy(dst_hbm.at[wid], dst_v)
  pltpu.sync_copy(ew_hbm.at[wid], ew_v)
  pltpu.sync_copy(dinv_hbm, dinv_v)

  def zzero(k, _):
    for f in range(D_HID // LANES):
      rows0_v[k, pl.ds(f * LANES, LANES)] = jnp.zeros((LANES,), jnp.float32)
    return _
  lax.fori_loop(0, CB1 * K, zzero, None)
  pltpu.sync_copy(rows0_v, acc.at[pl.ds(sid * ROWS_PER_TILE, CB1 * K)])
  pltpu.sync_copy(rows0_v.at[pl.ds(0, ROWS_PER_TILE - CB1 * K)],
                  acc.at[pl.ds(sid * ROWS_PER_TILE + CB1 * K,
                               ROWS_PER_TILE - CB1 * K)])

  def norm_it(i, _):
    sl = pl.ds(i * LANES, LANES)
    sv = src_v[sl]
    dv = dst_v[sl]
    nv = (plsc.load_gather(dinv_v, [sv]) * ew_v[sl]
          * plsc.load_gather(dinv_v, [dv]))
    norm_v[sl] = nv
    return _
  lax.fori_loop(0, C * (K // LANES), norm_it, None)
  pltpu.sync_copy(norm_v, norm_hbm.at[wid])

  plsc.subcore_barrier()

  _ring_loop(xw_hbm, src_v, dst_v, norm_v, rows, gsem, ssem, acc, D_HID, CB1)

  plsc.subcore_barrier()
  sl = pl.ds(sid * ROWS_PER_TILE, ROWS_PER_TILE)
  pltpu.sync_copy(acc.at[sl], hpart_hbm.at[cid, sl])


def _sc_prop1(xw, src3, dst3, ew3, dinv):
  mesh = plsc.VectorSubcoreMesh(core_axis_name="c", subcore_axis_name="s")
  fn = pl.kernel(
      _prop1_body,
      out_type=(
          jax.ShapeDtypeStruct((NC, N_PAD, D_HID), jnp.float32),
          jax.ShapeDtypeStruct((NW, C * K), jnp.float32),
      ),
      mesh=mesh,
      compiler_params=pltpu.CompilerParams(needs_layout_passes=False, use_tc_tiling_on_sc=False),
      scratch_types=[
          pltpu.VMEM((C * K,), jnp.int32),
          pltpu.VMEM((C * K,), jnp.int32),
          pltpu.VMEM((C * K,), jnp.float32),
          pltpu.VMEM((C * K,), jnp.float32),
          pltpu.VMEM((N_PAD,), jnp.float32),
          pltpu.VMEM((CB1 * K, D_HID), jnp.float32),
          pltpu.VMEM((CB1 * K, D_HID), jnp.float32),
          pltpu.SemaphoreType.DMA,
          pltpu.SemaphoreType.DMA,
          pltpu.SemaphoreType.DMA,
          pltpu.SemaphoreType.DMA,
          pltpu.VMEM_SHARED((N_PAD, D_HID), jnp.float32),
      ],
  )
  return fn(xw, src3, dst3, ew3, dinv)


def _prop2_body(hw_hbm, src_hbm, dst_hbm, norm_hbm, lpart_hbm,
                src_v, dst_v, norm_v,
                rows0_v, rows1_v,
                gsem0, gsem1, ssem0, ssem1, acc):
  cid = lax.axis_index("c")
  sid = lax.axis_index("s")
  wid = cid * NS + sid
  rows = (rows0_v, rows1_v)
  gsem = (gsem0, gsem1)
  ssem = (ssem0, ssem1)
  pltpu.sync_copy(src_hbm.at[wid], src_v)
  pltpu.sync_copy(dst_hbm.at[wid], dst_v)
  pltpu.sync_copy(norm_hbm.at[wid], norm_v)

  def zzero(k, _):
    rows0_v[k, pl.ds(0, LANES)] = jnp.zeros((LANES,), jnp.float32)
    return _
  lax.fori_loop(0, CB2 * K, zzero, None)
  pltpu.sync_copy(rows0_v.at[pl.ds(0, ROWS_PER_TILE)],
                  acc.at[pl.ds(sid * ROWS_PER_TILE, ROWS_PER_TILE)])

  plsc.subcore_barrier()

  _ring_loop(hw_hbm, src_v, dst_v, norm_v, rows, gsem, ssem, acc, D_OUT, CB2)

  plsc.subcore_barrier()
  sl = pl.ds(sid * ROWS_PER_TILE, ROWS_PER_TILE)
  pltpu.sync_copy(acc.at[sl], lpart_hbm.at[cid, sl])


def _sc_prop2(hw, src3, dst3, norm3):
  mesh = plsc.VectorSubcoreMesh(core_axis_name="c", subcore_axis_name="s")
  fn = pl.kernel(
      _prop2_body,
      out_type=jax.ShapeDtypeStruct((NC, N_PAD, D_OUT), jnp.float32),
      mesh=mesh,
      compiler_params=pltpu.CompilerParams(needs_layout_passes=False, use_tc_tiling_on_sc=False),
      scratch_types=[
          pltpu.VMEM((C * K,), jnp.int32),
          pltpu.VMEM((C * K,), jnp.int32),
          pltpu.VMEM((C * K,), jnp.float32),
          pltpu.VMEM((CB2 * K, D_OUT), jnp.float32),
          pltpu.VMEM((CB2 * K, D_OUT), jnp.float32),
          pltpu.SemaphoreType.DMA,
          pltpu.SemaphoreType.DMA,
          pltpu.SemaphoreType.DMA,
          pltpu.SemaphoreType.DMA,
          pltpu.VMEM_SHARED((N_PAD, D_OUT), jnp.float32),
      ],
  )
  return fn(hw, src3, dst3, norm3)


def _tc2_body(hpart, b1, w2, hw_o):
  h = jax.nn.relu(hpart[0] + hpart[1] + b1[...])
  hw_o[...] = jnp.dot(h, w2[...], preferred_element_type=jnp.float32)


def _tc3_body(lpart, b2, logits_o, soft_o):
  logits = (lpart[0] + lpart[1] + b2[...])[:N]
  logits_o[...] = logits
  m = jnp.max(logits, axis=1, keepdims=True)
  e = jnp.exp(logits - m)
  soft_o[...] = e / jnp.sum(e, axis=1, keepdims=True)


def kernel(x, edge_index, edge_weight, mu_w1, rho_w1, mu_b1, rho_b1,
           mu_w2, rho_w2, mu_b2, rho_b2):
  src = jnp.pad(edge_index[0], (0, E_PAD - E))
  dst = jnp.pad(edge_index[1], (0, E_PAD - E))
  ew = jnp.pad(edge_weight, (0, E_PAD - E))
  src2 = src.reshape(NW, EW_PAD)
  dst2 = dst.reshape(NW, EW_PAD)
  ew2 = ew.reshape(NW, EW_PAD)

  degp = _sc_deg(dst, ew)

  dinv2, xw1, logp, logq = pl.pallas_call(
      _tc1_body,
      out_shape=(
          jax.ShapeDtypeStruct((1, N_PAD), jnp.float32),
          jax.ShapeDtypeStruct((N, D_HID), jnp.float32),
          jax.ShapeDtypeStruct((1, 1), jnp.float32),
          jax.ShapeDtypeStruct((1, 1), jnp.float32),
      ),
  )(degp, x, mu_w1, rho_w1, mu_b1.reshape(1, D_HID), rho_b1.reshape(1, D_HID),
    mu_w2, rho_w2, mu_b2.reshape(1, D_OUT), rho_b2.reshape(1, D_OUT))

  hpart, norm3 = _sc_prop1(xw1, src2, dst2, ew2, dinv2.reshape(N_PAD))

  hw2 = pl.pallas_call(
      _tc2_body,
      out_shape=jax.ShapeDtypeStruct((N_PAD, D_OUT), jnp.float32),
  )(hpart, mu_b1.reshape(1, D_HID), mu_w2)

  lpart = _sc_prop2(hw2, src2, dst2, norm3)

  logits, soft = pl.pallas_call(
      _tc3_body,
      out_shape=(
          jax.ShapeDtypeStruct((N, D_OUT), jnp.float32),
          jax.ShapeDtypeStruct((N, D_OUT), jnp.float32),
      ),
  )(lpart, mu_b2.reshape(1, D_OUT))

  return logits, soft, logp[0, 0], logq[0, 0]


# speedup vs baseline: 1.3264x; 1.3264x over previous
import functools

import jax
import jax.numpy as jnp
from jax import lax
from jax.experimental import pallas as pl
from jax.experimental.pallas import tpu as pltpu
from jax.experimental.pallas import tpu_sc as plsc

N = 10000
E = 320000
D_IN = 128
D_HID = 64
D_OUT = 16
PI = 0.5
SIGMA_1 = 1.0
SIGMA_2 = 0.0025

NC = 2
NS = 16
NW = NC * NS
LANES = 16
K = 128
C = 2 * (-(-E // (NW * K * 2)))
EW_PAD = C * K
E_PAD = NW * EW_PAD
N_PAD = 10240
ROWS_PER_TILE = N_PAD // NS
C2 = C // 2


def _deg_body(dst_hbm, ew_hbm, degp_hbm, dst_v, ew_v, deg_v):
  cid = lax.axis_index("c")
  sid = lax.axis_index("s")
  wid = cid * NS + sid
  pltpu.sync_copy(dst_hbm.at[wid], dst_v)
  pltpu.sync_copy(ew_hbm.at[wid], ew_v)

  def zero(i, _):
    deg_v[pl.ds(i * LANES, LANES)] = jnp.zeros((LANES,), jnp.float32)
    return _
  lax.fori_loop(0, N_PAD // LANES, zero, None)

  def acc(i, _):
    sl = pl.ds(i * LANES, LANES)
    plsc.addupdate_scatter(deg_v, [dst_v[sl]], ew_v[sl])
    return _
  lax.fori_loop(0, EW_PAD // LANES, acc, None)
  pltpu.sync_copy(deg_v, degp_hbm.at[wid])


def _sc_deg(dst_flat, ew_flat):
  mesh = plsc.VectorSubcoreMesh(core_axis_name="c", subcore_axis_name="s")
  fn = pl.kernel(
      _deg_body,
      out_type=jax.ShapeDtypeStruct((NW, N_PAD), jnp.float32),
      mesh=mesh,
      compiler_params=pltpu.CompilerParams(needs_layout_passes=False, use_tc_tiling_on_sc=False),
      scratch_types=[
          pltpu.VMEM((EW_PAD,), jnp.int32),
          pltpu.VMEM((EW_PAD,), jnp.float32),
          pltpu.VMEM((N_PAD,), jnp.float32),
      ],
  )
  return fn(dst_flat.reshape(NW, EW_PAD), ew_flat.reshape(NW, EW_PAD))


def _log_gauss_const(sigma):
  return -0.5 * jnp.log(2.0 * jnp.pi) - jnp.log(sigma)


def _tc1_body(degp, x, w1, rho_w1, b1, rho_b1, w2, rho_w2, b2, rho_b2,
              dinv_o, xw_o, logp_o, logq_o):
  deg = jnp.sum(degp[...], axis=0)
  dinv = jnp.where(deg > 0, lax.rsqrt(deg), 0.0)
  dinv_o[...] = dinv[None, :]
  xw_o[...] = jnp.dot(x[...], w1[...], preferred_element_type=jnp.float32)

  def log_prior(w):
    lg1 = _log_gauss_const(SIGMA_1) - w * w / (2.0 * SIGMA_1 ** 2)
    lg2 = _log_gauss_const(SIGMA_2) - w * w / (2.0 * SIGMA_2 ** 2)
    p = PI * jnp.exp(lg1) + (1.0 - PI) * jnp.exp(lg2)
    return jnp.sum(jnp.log(p + 1e-30))

  def log_q(rho):
    s = jnp.log1p(jnp.exp(rho))
    return jnp.sum(-0.5 * jnp.log(2.0 * jnp.pi) - jnp.log(s))

  logp = (log_prior(w1[...]) + log_prior(b1[...])
          + log_prior(w2[...]) + log_prior(b2[...]))
  logq = (log_q(rho_w1[...]) + log_q(rho_b1[...])
          + log_q(rho_w2[...]) + log_q(rho_b2[...]))
  logp_o[...] = logp[None, None]
  logq_o[...] = logq[None, None]


def _ring_loop(table_sp, src_v, dst_v, norm_v, rows, gsem, ssem, acc, d,
               nchunks):
  pltpu.async_copy(table_sp.at[src_v.at[0]], rows[0], gsem[0])

  def cc_body(cc, _):
    for b in range(2):
      c = cc * 2 + b
      pb = 1 - b
      pltpu.make_async_copy(table_sp.at[src_v.at[c]], rows[b], gsem[b]).wait()

      @pl.when(c > 0)
      def _wait_prev_scatter():
        pltpu.make_async_copy(
            rows[pb], acc.at[dst_v.at[c - 1]], ssem[pb]).wait()

      @pl.when(c + 1 < nchunks)
      def _prefetch_next():
        pltpu.async_copy(table_sp.at[src_v.at[c + 1]], rows[pb], gsem[pb])

      def scale(j, _):
        nv = norm_v[c, pl.ds(j * LANES, LANES)]
        for l in range(LANES):
          k = j * LANES + l
          s = nv[l]
          for f in range(d // LANES):
            sl = pl.ds(f * LANES, LANES)
            rows[b][k, sl] = rows[b][k, sl] * s
        return _
      lax.fori_loop(0, K // LANES, scale, None)
      pltpu.async_copy(rows[b], acc.at[dst_v.at[c]], ssem[b], add=True)
    return _
  lax.fori_loop(0, nchunks // 2, cc_body, None)
  lb = (nchunks - 1) % 2
  pltpu.make_async_copy(
      rows[lb], acc.at[dst_v.at[nchunks - 1]], ssem[lb]).wait()


def _prop1_body(xw_hbm, src_hbm, dst_hbm, ew_hbm, dinv_hbm,
                hpart_hbm, norm_hbm,
                src_v, dst_v, ewn_v, dinv_v, rows0_v, rows1_v,
                gsem0, gsem1, ssem0, ssem1, table_sp, acc):
  cid = lax.axis_index("c")
  sid = lax.axis_index("s")
  wid = cid * NS + sid
  rows = (rows0_v, rows1_v)
  gsem = (gsem0, gsem1)
  ssem = (ssem0, ssem1)
  pltpu.sync_copy(dinv_hbm, dinv_v)

  pltpu.sync_copy(xw_hbm.at[pl.ds(sid * (N // NS), N // NS)],
                  table_sp.at[pl.ds(sid * (N // NS), N // NS)])

  def zzero(k, _):
    for f in range(D_HID // LANES):
      rows0_v[k, pl.ds(f * LANES, LANES)] = jnp.zeros((LANES,), jnp.float32)
    return _
  lax.fori_loop(0, K, zzero, None)
  for t in range(ROWS_PER_TILE // K):
    pltpu.sync_copy(rows0_v, acc.at[pl.ds(sid * ROWS_PER_TILE + t * K, K)])

  plsc.subcore_barrier()

  for h in range(2):
    hc = pl.ds(h * C2, C2)
    pltpu.sync_copy(src_hbm.at[wid, hc], src_v)
    pltpu.sync_copy(dst_hbm.at[wid, hc], dst_v)
    pltpu.sync_copy(ew_hbm.at[wid, hc], ewn_v)

    def norm_it(i, _):
      c = i // (K // LANES)
      j = i % (K // LANES)
      sl = pl.ds(j * LANES, LANES)
      nv = (plsc.load_gather(dinv_v, [src_v[c, sl]]) * ewn_v[c, sl]
            * plsc.load_gather(dinv_v, [dst_v[c, sl]]))
      ewn_v[c, sl] = nv
      return _
    lax.fori_loop(0, C2 * (K // LANES), norm_it, None)
    pltpu.sync_copy(ewn_v, norm_hbm.at[wid, hc])

    _ring_loop(table_sp, src_v, dst_v, ewn_v, rows, gsem, ssem, acc,
               D_HID, C2)

  plsc.subcore_barrier()
  sl = pl.ds(sid * ROWS_PER_TILE, ROWS_PER_TILE)
  pltpu.sync_copy(acc.at[sl], hpart_hbm.at[cid, sl])


def _sc_prop1(xw, src3, dst3, ew3, dinv):
  mesh = plsc.VectorSubcoreMesh(core_axis_name="c", subcore_axis_name="s")
  fn = pl.kernel(
      _prop1_body,
      out_type=(
          jax.ShapeDtypeStruct((NC, N_PAD, D_HID), jnp.float32),
          jax.ShapeDtypeStruct((NW, C, K), jnp.float32),
      ),
      mesh=mesh,
      compiler_params=pltpu.CompilerParams(needs_layout_passes=False, use_tc_tiling_on_sc=False),
      scratch_types=[
          pltpu.VMEM((C2, K), jnp.int32),
          pltpu.VMEM((C2, K), jnp.int32),
          pltpu.VMEM((C2, K), jnp.float32),
          pltpu.VMEM((N_PAD,), jnp.float32),
          pltpu.VMEM((K, D_HID), jnp.float32),
          pltpu.VMEM((K, D_HID), jnp.float32),
          pltpu.SemaphoreType.DMA,
          pltpu.SemaphoreType.DMA,
          pltpu.SemaphoreType.DMA,
          pltpu.SemaphoreType.DMA,
          pltpu.VMEM_SHARED((N_PAD, D_HID), jnp.float32),
          pltpu.VMEM_SHARED((N_PAD, D_HID), jnp.float32),
      ],
  )
  return fn(xw, src3, dst3, ew3, dinv)


def _prop2_body(hw_hbm, src_hbm, dst_hbm, norm_hbm, lpart_hbm,
                src_v, dst_v, norm_v, rows0_v, rows1_v,
                gsem0, gsem1, ssem0, ssem1, table_sp, acc):
  cid = lax.axis_index("c")
  sid = lax.axis_index("s")
  wid = cid * NS + sid
  rows = (rows0_v, rows1_v)
  gsem = (gsem0, gsem1)
  ssem = (ssem0, ssem1)
  pltpu.sync_copy(src_hbm.at[wid], src_v)
  pltpu.sync_copy(dst_hbm.at[wid], dst_v)
  pltpu.sync_copy(norm_hbm.at[wid], norm_v)

  pltpu.sync_copy(hw_hbm.at[pl.ds(sid * ROWS_PER_TILE, ROWS_PER_TILE)],
                  table_sp.at[pl.ds(sid * ROWS_PER_TILE, ROWS_PER_TILE)])

  def zzero(k, _):
    rows0_v[k, pl.ds(0, LANES)] = jnp.zeros((LANES,), jnp.float32)
    return _
  lax.fori_loop(0, K, zzero, None)
  for t in range(ROWS_PER_TILE // K):
    pltpu.sync_copy(rows0_v, acc.at[pl.ds(sid * ROWS_PER_TILE + t * K, K)])

  plsc.subcore_barrier()

  _ring_loop(table_sp, src_v, dst_v, norm_v, rows, gsem, ssem, acc, D_OUT, C)

  plsc.subcore_barrier()
  sl = pl.ds(sid * ROWS_PER_TILE, ROWS_PER_TILE)
  pltpu.sync_copy(acc.at[sl], lpart_hbm.at[cid, sl])


def _sc_prop2(hw, src3, dst3, norm3):
  mesh = plsc.VectorSubcoreMesh(core_axis_name="c", subcore_axis_name="s")
  fn = pl.kernel(
      _prop2_body,
      out_type=jax.ShapeDtypeStruct((NC, N_PAD, D_OUT), jnp.float32),
      mesh=mesh,
      compiler_params=pltpu.CompilerParams(needs_layout_passes=False, use_tc_tiling_on_sc=False),
      scratch_types=[
          pltpu.VMEM((C, K), jnp.int32),
          pltpu.VMEM((C, K), jnp.int32),
          pltpu.VMEM((C, K), jnp.float32),
          pltpu.VMEM((K, D_OUT), jnp.float32),
          pltpu.VMEM((K, D_OUT), jnp.float32),
          pltpu.SemaphoreType.DMA,
          pltpu.SemaphoreType.DMA,
          pltpu.SemaphoreType.DMA,
          pltpu.SemaphoreType.DMA,
          pltpu.VMEM_SHARED((N_PAD, D_OUT), jnp.float32),
          pltpu.VMEM_SHARED((N_PAD, D_OUT), jnp.float32),
      ],
  )
  return fn(hw, src3, dst3, norm3)


def _tc2_body(hpart, b1, w2, hw_o):
  h = jax.nn.relu(hpart[0] + hpart[1] + b1[...])
  hw_o[...] = jnp.dot(h, w2[...], preferred_element_type=jnp.float32)


def _tc3_body(lpart, b2, logits_o, soft_o):
  logits = (lpart[0] + lpart[1] + b2[...])[:N]
  logits_o[...] = logits
  m = jnp.max(logits, axis=1, keepdims=True)
  e = jnp.exp(logits - m)
  soft_o[...] = e / jnp.sum(e, axis=1, keepdims=True)


def kernel(x, edge_index, edge_weight, mu_w1, rho_w1, mu_b1, rho_b1,
           mu_w2, rho_w2, mu_b2, rho_b2):
  src = jnp.pad(edge_index[0], (0, E_PAD - E))
  dst = jnp.pad(edge_index[1], (0, E_PAD - E))
  ew = jnp.pad(edge_weight, (0, E_PAD - E))
  src3 = src.reshape(NW, C, K)
  dst3 = dst.reshape(NW, C, K)
  ew3 = ew.reshape(NW, C, K)

  degp = _sc_deg(dst, ew)

  dinv2, xw1, logp, logq = pl.pallas_call(
      _tc1_body,
      out_shape=(
          jax.ShapeDtypeStruct((1, N_PAD), jnp.float32),
          jax.ShapeDtypeStruct((N, D_HID), jnp.float32),
          jax.ShapeDtypeStruct((1, 1), jnp.float32),
          jax.ShapeDtypeStruct((1, 1), jnp.float32),
      ),
  )(degp, x, mu_w1, rho_w1, mu_b1.reshape(1, D_HID), rho_b1.reshape(1, D_HID),
    mu_w2, rho_w2, mu_b2.reshape(1, D_OUT), rho_b2.reshape(1, D_OUT))

  hpart, norm3 = _sc_prop1(xw1, src3, dst3, ew3, dinv2.reshape(N_PAD))

  hw2 = pl.pallas_call(
      _tc2_body,
      out_shape=jax.ShapeDtypeStruct((N_PAD, D_OUT), jnp.float32),
  )(hpart, mu_b1.reshape(1, D_HID), mu_w2)

  lpart = _sc_prop2(hw2, src3, dst3, norm3)

  logits, soft = pl.pallas_call(
      _tc3_body,
      out_shape=(
          jax.ShapeDtypeStruct((N, D_OUT), jnp.float32),
          jax.ShapeDtypeStruct((N, D_OUT), jnp.float32),
      ),
  )(lpart, mu_b2.reshape(1, D_OUT))

  return logits, soft, logp[0, 0], logq[0, 0]
